# baseline (speedup 1.0000x reference)
import functools

import jax
import jax.numpy as jnp
from jax import lax
from jax.experimental import pallas as pl
from jax.experimental.pallas import tpu as pltpu

NUM_LAYERS = 4
NUM_EMBEDDINGS = 1024
EMBEDDING_DIM = 64
COMMITMENT_COST = 0.25

TILE = 1024


def _rvq_tile(x_ref, emb_ref, q_ref, idx_ref, loss_ref):
    r = x_ref[...]
    qacc = jnp.zeros_like(r)
    loss_acc = jnp.float32(0.0)
    for l in range(NUM_LAYERS):
        emb = emb_ref[l]
        e_norms = jnp.sum(emb * emb, axis=1)
        r_norms = jnp.sum(r * r, axis=1, keepdims=True)
        dots = lax.dot_general(
            r, emb, (((1,), (1,)), ((), ())),
            preferred_element_type=jnp.float32,
        )
        dist = (r_norms + e_norms[None, :]) - 2.0 * dots
        mins = jnp.min(dist, axis=1, keepdims=True)
        jidx = lax.broadcasted_iota(jnp.int32, dist.shape, 1)
        idx = jnp.min(
            jnp.where(dist == mins, jidx, NUM_EMBEDDINGS), axis=1
        )
        onehot = (jidx == idx[:, None]).astype(jnp.float32)
        q = lax.dot_general(
            onehot, emb, (((1,), (0,)), ((), ())),
            preferred_element_type=jnp.float32,
        )
        loss_acc += jnp.sum((q - r) * (q - r))
        q_ste = r + (q - r)
        r = r - q_ste
        qacc = qacc + q_ste
        idx_ref[l, :] = idx
    q_ref[...] = qacc
    loss_ref[...] = loss_acc.reshape(1, 1, 1)


@functools.partial(jax.jit, static_argnames=())
def kernel(x, embeddings):
    B, S, D = x.shape
    n_tokens = B * S
    x_flat = x.reshape(n_tokens, D)
    grid = (n_tokens // TILE,)

    q_flat, idx_lt, loss = pl.pallas_call(
        _rvq_tile,
        grid=grid,
        in_specs=[
            pl.BlockSpec((TILE, D), lambda i: (i, 0)),
            pl.BlockSpec((NUM_LAYERS, NUM_EMBEDDINGS, D), lambda i: (0, 0, 0)),
        ],
        out_specs=[
            pl.BlockSpec((TILE, D), lambda i: (i, 0)),
            pl.BlockSpec((NUM_LAYERS, TILE), lambda i: (0, i)),
            pl.BlockSpec((1, 1, 1), lambda i: (i, 0, 0)),
        ],
        out_shape=[
            jax.ShapeDtypeStruct((n_tokens, D), jnp.float32),
            jax.ShapeDtypeStruct((NUM_LAYERS, n_tokens), jnp.int32),
            jax.ShapeDtypeStruct((grid[0], 1, 1), jnp.float32),
        ],
        compiler_params=pltpu.CompilerParams(
            dimension_semantics=("parallel",),
        ),
    )(x_flat, embeddings)

    quantized_out = q_flat.reshape(B, S, D)
    losses = jnp.sum(loss) * (COMMITMENT_COST / n_tokens / D)
    all_indices = idx_lt.T.reshape(B, S, NUM_LAYERS)
    return quantized_out, losses, all_indices


# speedup vs baseline: 1.1009x; 1.1009x over previous
import functools

import jax
import jax.numpy as jnp
from jax import lax
from jax.experimental import pallas as pl
from jax.experimental.pallas import tpu as pltpu

NUM_LAYERS = 4
NUM_EMBEDDINGS = 1024
EMBEDDING_DIM = 64
COMMITMENT_COST = 0.25

TILE = 1024


def _rvq_tile(x_ref, emb_ref, q_ref, idx_ref, loss_ref, en_ref, e2_ref):
    i = pl.program_id(0)

    @pl.when(i == 0)
    def _precompute():
        for l in range(NUM_LAYERS):
            emb = emb_ref[l]
            en_ref[l, :] = jnp.sum(emb * emb, axis=1)
            e2_ref[l] = emb + emb

    r = x_ref[...]
    qacc = jnp.zeros_like(r)
    loss_acc = jnp.float32(0.0)
    for l in range(NUM_LAYERS):
        emb = emb_ref[l]
        e_norms = en_ref[l, :]
        r_norms = jnp.sum(r * r, axis=1, keepdims=True)
        dots2 = lax.dot_general(
            r, e2_ref[l], (((1,), (1,)), ((), ())),
            preferred_element_type=jnp.float32,
        )
        dist = (r_norms + e_norms[None, :]) - dots2
        mins = jnp.min(dist, axis=1, keepdims=True)
        jidx_f = lax.broadcasted_iota(
            jnp.int32, (1, NUM_EMBEDDINGS), 1
        ).astype(jnp.float32)
        idx_f = jnp.min(
            jnp.where(dist == mins, jidx_f, jnp.float32(NUM_EMBEDDINGS)),
            axis=1,
        )
        idx = idx_f.astype(jnp.int32)
        onehot = (jidx_f == idx_f[:, None]).astype(jnp.float32)
        q = lax.dot_general(
            onehot, emb, (((1,), (0,)), ((), ())),
            preferred_element_type=jnp.float32,
        )
        loss_acc += jnp.sum((q - r) * (q - r))
        q_ste = r + (q - r)
        r = r - q_ste
        qacc = qacc + q_ste
        idx_ref[l, :] = idx
    q_ref[...] = qacc
    loss_ref[...] = loss_acc.reshape(1, 1, 1)


@functools.partial(jax.jit, static_argnames=())
def kernel(x, embeddings):
    B, S, D = x.shape
    n_tokens = B * S
    x_flat = x.reshape(n_tokens, D)
    grid = (n_tokens // TILE,)

    q_flat, idx_lt, loss = pl.pallas_call(
        _rvq_tile,
        grid=grid,
        in_specs=[
            pl.BlockSpec((TILE, D), lambda i: (i, 0)),
            pl.BlockSpec((NUM_LAYERS, NUM_EMBEDDINGS, D), lambda i: (0, 0, 0)),
        ],
        out_specs=[
            pl.BlockSpec((TILE, D), lambda i: (i, 0)),
            pl.BlockSpec((NUM_LAYERS, TILE), lambda i: (0, i)),
            pl.BlockSpec((1, 1, 1), lambda i: (i, 0, 0)),
        ],
        out_shape=[
            jax.ShapeDtypeStruct((n_tokens, D), jnp.float32),
            jax.ShapeDtypeStruct((NUM_LAYERS, n_tokens), jnp.int32),
            jax.ShapeDtypeStruct((grid[0], 1, 1), jnp.float32),
        ],
        scratch_shapes=[
            pltpu.VMEM((NUM_LAYERS, NUM_EMBEDDINGS), jnp.float32),
            pltpu.VMEM((NUM_LAYERS, NUM_EMBEDDINGS, EMBEDDING_DIM), jnp.float32),
        ],
    )(x_flat, embeddings)

    quantized_out = q_flat.reshape(B, S, D)
    losses = jnp.sum(loss) * (COMMITMENT_COST / n_tokens / D)
    all_indices = idx_lt.T.reshape(B, S, NUM_LAYERS)
    return quantized_out, losses, all_indices


# speedup vs baseline: 1.1758x; 1.0681x over previous
import functools

import jax
import jax.numpy as jnp
from jax import lax
from jax.experimental import pallas as pl
from jax.experimental.pallas import tpu as pltpu

NUM_LAYERS = 4
NUM_EMBEDDINGS = 1024
EMBEDDING_DIM = 64
COMMITMENT_COST = 0.25

TILE = 2048


def _rvq_tile(x_ref, emb_ref, q_ref, idx_ref, loss_ref, en_ref, e2_ref):
    i = pl.program_id(0)

    @pl.when(i == 0)
    def _precompute():
        for l in range(NUM_LAYERS):
            emb = emb_ref[l]
            en_ref[l, :] = jnp.sum(emb * emb, axis=1)
            e2_ref[l] = emb + emb

    r = x_ref[...]
    qacc = jnp.zeros_like(r)
    loss_acc = jnp.float32(0.0)
    for l in range(NUM_LAYERS):
        emb = emb_ref[l]
        e_norms = en_ref[l, :]
        r_norms = jnp.sum(r * r, axis=1, keepdims=True)
        dots2 = lax.dot_general(
            r, e2_ref[l], (((1,), (1,)), ((), ())),
            preferred_element_type=jnp.float32,
        )
        dist = (r_norms + e_norms[None, :]) - dots2
        mins = jnp.min(dist, axis=1, keepdims=True)
        jidx_f = lax.broadcasted_iota(
            jnp.int32, (1, NUM_EMBEDDINGS), 1
        ).astype(jnp.float32)
        idx_f = jnp.min(
            jnp.where(dist == mins, jidx_f, jnp.float32(NUM_EMBEDDINGS)),
            axis=1,
        )
        idx = idx_f.astype(jnp.int32)
        onehot = (jidx_f == idx_f[:, None]).astype(jnp.float32)
        q = lax.dot_general(
            onehot, emb, (((1,), (0,)), ((), ())),
            preferred_element_type=jnp.float32,
        )
        loss_acc += jnp.sum((q - r) * (q - r))
        q_ste = r + (q - r)
        r = r - q_ste
        qacc = qacc + q_ste
        idx_ref[l, :] = idx
    q_ref[...] = qacc
    loss_ref[...] = loss_acc.reshape(1, 1, 1)


@functools.partial(jax.jit, static_argnames=())
def kernel(x, embeddings):
    B, S, D = x.shape
    n_tokens = B * S
    x_flat = x.reshape(n_tokens, D)
    grid = (n_tokens // TILE,)

    q_flat, idx_lt, loss = pl.pallas_call(
        _rvq_tile,
        grid=grid,
        in_specs=[
            pl.BlockSpec((TILE, D), lambda i: (i, 0)),
            pl.BlockSpec((NUM_LAYERS, NUM_EMBEDDINGS, D), lambda i: (0, 0, 0)),
        ],
        out_specs=[
            pl.BlockSpec((TILE, D), lambda i: (i, 0)),
            pl.BlockSpec((NUM_LAYERS, TILE), lambda i: (0, i)),
            pl.BlockSpec((1, 1, 1), lambda i: (i, 0, 0)),
        ],
        out_shape=[
            jax.ShapeDtypeStruct((n_tokens, D), jnp.float32),
            jax.ShapeDtypeStruct((NUM_LAYERS, n_tokens), jnp.int32),
            jax.ShapeDtypeStruct((grid[0], 1, 1), jnp.float32),
        ],
        scratch_shapes=[
            pltpu.VMEM((NUM_LAYERS, NUM_EMBEDDINGS), jnp.float32),
            pltpu.VMEM((NUM_LAYERS, NUM_EMBEDDINGS, EMBEDDING_DIM), jnp.float32),
        ],
    )(x_flat, embeddings)

    quantized_out = q_flat.reshape(B, S, D)
    losses = jnp.sum(loss) * (COMMITMENT_COST / n_tokens / D)
    all_indices = idx_lt.T.reshape(B, S, NUM_LAYERS)
    return quantized_out, losses, all_indices
